# baseline (speedup 1.0000x reference)
import functools

import jax
import jax.numpy as jnp
from jax import lax
from jax.experimental import pallas as pl
from jax.experimental.pallas import tpu as pltpu
from jax.experimental.pallas import tpu_sc as plsc

N = 10000
E = 160000
HID = 256
OUT_CH = 64

NC = 2
NS = 16
CK = 128
CP = 80
EPS = CP * CK
EPAD = NS * EPS
NDP = 10112
DRAIN = 624
ZR = NDP // NS


def _leaky(x):
    return jnp.where(x >= 0.0, x, 0.2 * x)


def _segsum_body(h2, src2, dst3, seg_out, sidx, didx, rows, acc, sem):
    c = lax.axis_index("c")
    s = lax.axis_index("s")

    pltpu.sync_copy(src2.at[c, s], sidx)
    pltpu.sync_copy(dst3.at[s], didx)

    def _fill_rows(i, _):
        for k in range(8):
            rows[i, pl.ds(k * 16, 16)] = jnp.zeros((16,), jnp.float32)
        return 0
    lax.fori_loop(0, CK, _fill_rows, 0)

    b0 = s * ZR
    for t in range(4):
        pltpu.sync_copy(rows, acc.at[pl.ds(b0 + t * CK, CK)])
    pltpu.sync_copy(rows.at[pl.ds(0, ZR - 4 * CK)],
                    acc.at[pl.ds(b0 + 4 * CK, ZR - 4 * CK)])
    plsc.subcore_barrier()

    def _step(j, _):
        pltpu.async_copy(h2.at[sidx.at[j]], rows, sem).wait()
        pltpu.sync_copy(rows, acc.at[didx.at[j]], add=True)
        return 0
    lax.fori_loop(0, CP, _step, 0)

    plsc.subcore_barrier()

    d0 = s * DRAIN
    pltpu.sync_copy(acc.at[pl.ds(d0, DRAIN)], seg_out.at[c, pl.ds(d0, DRAIN)])

    tail0 = NS * DRAIN
    tail = N - tail0

    @pl.when(s == NS - 1)
    def _():
        pltpu.sync_copy(acc.at[pl.ds(tail0, tail)],
                        seg_out.at[c, pl.ds(tail0, tail)])


@functools.lru_cache(maxsize=None)
def _segsum_call():
    return pl.kernel(
        _segsum_body,
        out_type=jax.ShapeDtypeStruct((NC, N, 128), jnp.float32),
        mesh=plsc.VectorSubcoreMesh(core_axis_name="c", subcore_axis_name="s",
                                    num_cores=NC, num_subcores=NS),
        scratch_types=[
            pltpu.VMEM((CP, CK), jnp.int32),
            pltpu.VMEM((CP, CK), jnp.int32),
            pltpu.VMEM((CK, 128), jnp.float32),
            pltpu.VMEM_SHARED((NDP, 128), jnp.float32),
            pltpu.SemaphoreType.DMA,
        ],
    )


def _segsum(h, src2, dst3):
    h2 = h.reshape(2 * N, 128)
    return _segsum_call()(h2, src2, dst3)


def _count_body(dstall, cnt_out, didx, onesb, cacc):
    c = lax.axis_index("c")
    s = lax.axis_index("s")

    def _fill(i, _):
        for k in range(8):
            onesb[i, pl.ds(k * 16, 16)] = jnp.zeros((16,), jnp.float32)
        return 0
    lax.fori_loop(0, CK, _fill, 0)

    b0 = s * ZR
    for t in range(4):
        pltpu.sync_copy(onesb, cacc.at[pl.ds(b0 + t * CK, CK)])
    pltpu.sync_copy(onesb.at[pl.ds(0, ZR - 4 * CK)],
                    cacc.at[pl.ds(b0 + 4 * CK, ZR - 4 * CK)])
    plsc.subcore_barrier()

    for k in range(3):
        pltpu.sync_copy(dstall.at[k, s], didx)

        def _lane(i, _, k=k):
            if k > 0:
                onesb[i, pl.ds(16 * (k - 1), 16)] = jnp.zeros((16,),
                                                              jnp.float32)
            onesb[i, pl.ds(16 * k, 16)] = jnp.ones((16,), jnp.float32)
            return 0
        lax.fori_loop(0, CK, _lane, 0)

        def _step(t, _):
            pltpu.sync_copy(onesb, cacc.at[didx.at[2 * t + c]], add=True)
            return 0
        lax.fori_loop(0, CP // 2, _step, 0)

    plsc.subcore_barrier()

    d0 = s * DRAIN
    pltpu.sync_copy(cacc.at[pl.ds(d0, DRAIN)], cnt_out.at[c, pl.ds(d0, DRAIN)])

    tail0 = NS * DRAIN

    @pl.when(s == NS - 1)
    def _():
        pltpu.sync_copy(cacc.at[pl.ds(tail0, N - tail0)],
                        cnt_out.at[c, pl.ds(tail0, N - tail0)])


@functools.lru_cache(maxsize=None)
def _count_call():
    return pl.kernel(
        _count_body,
        out_type=jax.ShapeDtypeStruct((NC, N, 128), jnp.float32),
        mesh=plsc.VectorSubcoreMesh(core_axis_name="c", subcore_axis_name="s",
                                    num_cores=NC, num_subcores=NS),
        scratch_types=[
            pltpu.VMEM((CP, CK), jnp.int32),
            pltpu.VMEM((CK, 128), jnp.float32),
            pltpu.VMEM_SHARED((NDP, 128), jnp.float32),
        ],
    )


def _count3(dst_tt, dst_rr, dst_rt):
    parts = _count_call()(jnp.stack([dst_tt, dst_rr, dst_rt]))
    both = parts[0, :, 0:48] + parts[1, :, 0:48]
    return both[:, 0:16], both[:, 16:32], both[:, 32:48]


def _prep_edges(ei):
    src = ei[0]
    dst = ei[1]
    pad = EPAD - E
    srcp = jnp.concatenate([src, jnp.zeros((pad,), jnp.int32)])
    dstp = jnp.concatenate(
        [dst, N + (jnp.arange(pad, dtype=jnp.int32) % (NDP - N))])
    src2 = jnp.stack([2 * srcp, 2 * srcp + 1]).reshape(NC, NS, CP, CK)
    dst3 = dstp.reshape(NS, CP, CK)
    return src2, dst3


RB = 1000


def _post_body(x_ref, w1_ref, w2_ref, g_ref, b_ref, o_ref):
    h = _leaky(jnp.dot(x_ref[...], w1_ref[...],
                       preferred_element_type=jnp.float32))
    h = jnp.dot(h, w2_ref[...], preferred_element_type=jnp.float32)
    m = jnp.mean(h, axis=1, keepdims=True)
    v = jnp.mean((h - m) * (h - m), axis=1, keepdims=True)
    h = (h - m) * lax.rsqrt(v + 1e-5) * g_ref[...] + b_ref[...]
    o_ref[...] = _leaky(h)


def _post_tc(x, w1, w2, g, b):
    k = x.shape[1]
    return pl.pallas_call(
        _post_body,
        grid=(N // RB,),
        in_specs=[
            pl.BlockSpec((RB, k), lambda i: (i, 0)),
            pl.BlockSpec((k, HID), lambda i: (0, 0)),
            pl.BlockSpec((HID, HID), lambda i: (0, 0)),
            pl.BlockSpec((1, HID), lambda i: (0, 0)),
            pl.BlockSpec((1, HID), lambda i: (0, 0)),
        ],
        out_specs=pl.BlockSpec((RB, HID), lambda i: (i, 0)),
        out_shape=jax.ShapeDtypeStruct((N, HID), jnp.float32),
    )(x, w1, w2, g.reshape(1, HID), b.reshape(1, HID))


def _sage2_body(sa_ref, ca_ref, sb_ref, cb_ref, h_ref,
                wla_ref, wlb_ref, wr_ref, bb_ref, o_ref):
    ra = 1.0 / jnp.maximum(ca_ref[...][:, 0:1], 1.0)
    rb = 1.0 / jnp.maximum(cb_ref[...][:, 0:1], 1.0)
    acc = jnp.dot(h_ref[...], wr_ref[...], preferred_element_type=jnp.float32)
    acc += jnp.dot(sa_ref[0] * ra, wla_ref[0],
                   preferred_element_type=jnp.float32)
    acc += jnp.dot(sa_ref[1] * ra, wla_ref[1],
                   preferred_element_type=jnp.float32)
    acc += jnp.dot(sb_ref[0] * rb, wlb_ref[0],
                   preferred_element_type=jnp.float32)
    acc += jnp.dot(sb_ref[1] * rb, wlb_ref[1],
                   preferred_element_type=jnp.float32)
    o_ref[...] = _leaky(acc + bb_ref[...])


def _sage2_tc(sa, ca, sb, cb, h, wla, wlb, wr, bb):
    return pl.pallas_call(
        _sage2_body,
        grid=(N // RB,),
        in_specs=[
            pl.BlockSpec((NC, RB, 128), lambda i: (0, i, 0)),
            pl.BlockSpec((RB, 16), lambda i: (i, 0)),
            pl.BlockSpec((NC, RB, 128), lambda i: (0, i, 0)),
            pl.BlockSpec((RB, 16), lambda i: (i, 0)),
            pl.BlockSpec((RB, HID), lambda i: (i, 0)),
            pl.BlockSpec((NC, 128, HID), lambda i: (0, 0, 0)),
            pl.BlockSpec((NC, 128, HID), lambda i: (0, 0, 0)),
            pl.BlockSpec((HID, HID), lambda i: (0, 0)),
            pl.BlockSpec((1, HID), lambda i: (0, 0)),
        ],
        out_specs=pl.BlockSpec((RB, HID), lambda i: (i, 0)),
        out_shape=jax.ShapeDtypeStruct((N, HID), jnp.float32),
    )(sa, ca, sb, cb, h, wla, wlb, wr, bb)


def _sage1_body(sa_ref, ca_ref, h_ref, wla_ref, wr_ref, bb_ref, o_ref):
    ra = 1.0 / jnp.maximum(ca_ref[...][:, 0:1], 1.0)
    acc = jnp.dot(h_ref[...], wr_ref[...], preferred_element_type=jnp.float32)
    acc += jnp.dot(sa_ref[0] * ra, wla_ref[0],
                   preferred_element_type=jnp.float32)
    acc += jnp.dot(sa_ref[1] * ra, wla_ref[1],
                   preferred_element_type=jnp.float32)
    o_ref[...] = _leaky(acc + bb_ref[...])


def _sage1_tc(sa, ca, h, wla, wr, bb):
    return pl.pallas_call(
        _sage1_body,
        grid=(N // RB,),
        in_specs=[
            pl.BlockSpec((NC, RB, 128), lambda i: (0, i, 0)),
            pl.BlockSpec((RB, 16), lambda i: (i, 0)),
            pl.BlockSpec((RB, HID), lambda i: (i, 0)),
            pl.BlockSpec((NC, 128, HID), lambda i: (0, 0, 0)),
            pl.BlockSpec((HID, HID), lambda i: (0, 0)),
            pl.BlockSpec((1, HID), lambda i: (0, 0)),
        ],
        out_specs=pl.BlockSpec((RB, HID), lambda i: (i, 0)),
        out_shape=jax.ShapeDtypeStruct((N, HID), jnp.float32),
    )(sa, ca, h, wla, wr, bb)


@jax.jit
def kernel(x_target, x_reference, edge_index_tt, edge_index_rr,
           edge_index_rt, params):
    p = params
    e_tt = _prep_edges(edge_index_tt)
    e_rr = _prep_edges(edge_index_rr)
    e_rt = _prep_edges(edge_index_rt)
    cnt_tt, cnt_rr, cnt_rt = _count3(e_tt[1], e_rr[1], e_rt[1])

    h_tgt = _post_tc(x_target, p['W_win'], p['W_post'], p['ln_g'], p['ln_b'])
    h_ref = _post_tc(x_reference, p['W_exp'], p['W_post'], p['ln_g'], p['ln_b'])

    for layer in p['layers']:
        seg_tt = _segsum(h_tgt, *e_tt)
        seg_rr = _segsum(h_ref, *e_rr)
        seg_rt = _segsum(h_ref, *e_rt)

        wla = (0.5 * layer['Wl_tt']).reshape(NC, 128, HID)
        wlb = (0.5 * layer['Wl_rt']).reshape(NC, 128, HID)
        wr = 0.5 * (layer['Wr_tt'] + layer['Wr_rt'])
        bb = (0.5 * (layer['b_tt'] + layer['b_rt'])).reshape(1, HID)
        h_tgt_new = _sage2_tc(seg_tt, cnt_tt, seg_rt, cnt_rt, h_tgt,
                              wla, wlb, wr, bb)

        wlr = layer['Wl_rr'].reshape(NC, 128, HID)
        h_ref = _sage1_tc(seg_rr, cnt_rr, h_ref, wlr, layer['Wr_rr'],
                          layer['b_rr'].reshape(1, HID))
        h_tgt = h_tgt_new

    return (h_tgt, h_ref)


# speedup vs baseline: 1.3889x; 1.3889x over previous
import functools

import jax
import jax.numpy as jnp
from jax import lax
from jax.experimental import pallas as pl
from jax.experimental.pallas import tpu as pltpu
from jax.experimental.pallas import tpu_sc as plsc

N = 10000
E = 160000
HID = 256
OUT_CH = 64

NC = 2
NS = 16
CK = 128
CP = 79
EPS = CP * CK
EPAD = NS * EPS
NDP = 10112
DRAIN = 624
ZR = NDP // NS


def _leaky(x):
    return jnp.where(x >= 0.0, x, 0.2 * x)


def _segsum_body(h2, src2, dst3, seg_out, sidx, didx, rows, acc, sem):
    c = lax.axis_index("c")
    s = lax.axis_index("s")

    pltpu.sync_copy(src2.at[c, s], sidx)
    pltpu.sync_copy(dst3.at[s], didx)

    def _fill_rows(i, _):
        for k in range(8):
            rows[i, pl.ds(k * 16, 16)] = jnp.zeros((16,), jnp.float32)
        return 0
    lax.fori_loop(0, CK, _fill_rows, 0)

    b0 = s * ZR
    for t in range(4):
        pltpu.sync_copy(rows, acc.at[pl.ds(b0 + t * CK, CK)])
    pltpu.sync_copy(rows.at[pl.ds(0, ZR - 4 * CK)],
                    acc.at[pl.ds(b0 + 4 * CK, ZR - 4 * CK)])
    plsc.subcore_barrier()

    def _step(j, _):
        pltpu.async_copy(h2.at[sidx.at[j]], rows, sem).wait()
        pltpu.sync_copy(rows, acc.at[didx.at[j]], add=True)
        return 0
    lax.fori_loop(0, CP, _step, 0)

    plsc.subcore_barrier()

    d0 = s * DRAIN
    pltpu.sync_copy(acc.at[pl.ds(d0, DRAIN)], seg_out.at[c, pl.ds(d0, DRAIN)])

    tail0 = NS * DRAIN
    tail = N - tail0

    @pl.when(s == NS - 1)
    def _():
        pltpu.sync_copy(acc.at[pl.ds(tail0, tail)],
                        seg_out.at[c, pl.ds(tail0, tail)])


@functools.lru_cache(maxsize=None)
def _segsum_call():
    return pl.kernel(
        _segsum_body,
        out_type=jax.ShapeDtypeStruct((NC, N, 128), jnp.float32),
        mesh=plsc.VectorSubcoreMesh(core_axis_name="c", subcore_axis_name="s",
                                    num_cores=NC, num_subcores=NS),
        scratch_types=[
            pltpu.VMEM((CP, CK), jnp.int32),
            pltpu.VMEM((CP, CK), jnp.int32),
            pltpu.VMEM((CK, 128), jnp.float32),
            pltpu.VMEM_SHARED((NDP, 128), jnp.float32),
            pltpu.SemaphoreType.DMA,
        ],
    )


def _segsum(h, src2, dst3):
    h2 = h.reshape(2 * N, 128)
    return _segsum_call()(h2, src2, dst3)


def _count_body(dstall, cnt_out, didx, onesb, cacc):
    c = lax.axis_index("c")
    s = lax.axis_index("s")

    def _fill(i, _):
        for k in range(8):
            onesb[i, pl.ds(k * 16, 16)] = jnp.zeros((16,), jnp.float32)
        return 0
    lax.fori_loop(0, CK, _fill, 0)

    b0 = s * ZR
    for t in range(4):
        pltpu.sync_copy(onesb, cacc.at[pl.ds(b0 + t * CK, CK)])
    pltpu.sync_copy(onesb.at[pl.ds(0, ZR - 4 * CK)],
                    cacc.at[pl.ds(b0 + 4 * CK, ZR - 4 * CK)])
    plsc.subcore_barrier()

    for k in range(3):
        pltpu.sync_copy(dstall.at[k, s], didx)

        def _lane(i, _, k=k):
            if k > 0:
                onesb[i, pl.ds(16 * (k - 1), 16)] = jnp.zeros((16,),
                                                              jnp.float32)
            onesb[i, pl.ds(16 * k, 16)] = jnp.ones((16,), jnp.float32)
            return 0
        lax.fori_loop(0, CK, _lane, 0)

        def _step(t, _):
            pltpu.sync_copy(onesb, cacc.at[didx.at[2 * t + c]], add=True)
            return 0
        lax.fori_loop(0, CP // 2, _step, 0)

        if CP % 2:
            @pl.when(c == 1)
            def _():
                pltpu.sync_copy(onesb, cacc.at[didx.at[CP - 1]], add=True)

    plsc.subcore_barrier()

    d0 = s * DRAIN
    pltpu.sync_copy(cacc.at[pl.ds(d0, DRAIN)], cnt_out.at[c, pl.ds(d0, DRAIN)])

    tail0 = NS * DRAIN

    @pl.when(s == NS - 1)
    def _():
        pltpu.sync_copy(cacc.at[pl.ds(tail0, N - tail0)],
                        cnt_out.at[c, pl.ds(tail0, N - tail0)])


@functools.lru_cache(maxsize=None)
def _count_call():
    return pl.kernel(
        _count_body,
        out_type=jax.ShapeDtypeStruct((NC, N, 128), jnp.float32),
        mesh=plsc.VectorSubcoreMesh(core_axis_name="c", subcore_axis_name="s",
                                    num_cores=NC, num_subcores=NS),
        scratch_types=[
            pltpu.VMEM((CP, CK), jnp.int32),
            pltpu.VMEM((CK, 128), jnp.float32),
            pltpu.VMEM_SHARED((NDP, 128), jnp.float32),
        ],
    )


def _count3(dst_tt, dst_rr, dst_rt):
    parts = _count_call()(jnp.stack([dst_tt, dst_rr, dst_rt]))
    both = parts[0, :, 0:48] + parts[1, :, 0:48]
    return both[:, 0:16], both[:, 16:32], both[:, 32:48]


def _prep_edges(ei):
    src = ei[0]
    dst = ei[1]
    pad = EPAD - E
    srcp = jnp.concatenate([src, jnp.zeros((pad,), jnp.int32)])
    dstp = jnp.concatenate(
        [dst, N + (jnp.arange(pad, dtype=jnp.int32) % (NDP - N))])
    src2 = jnp.stack([2 * srcp, 2 * srcp + 1]).reshape(NC, NS, CP, CK)
    dst3 = dstp.reshape(NS, CP, CK)
    return src2, dst3


RB = 1000


def _post_body(x_ref, w1_ref, w2_ref, g_ref, b_ref, o_ref):
    h = _leaky(jnp.dot(x_ref[...], w1_ref[...],
                       preferred_element_type=jnp.float32))
    h = jnp.dot(h, w2_ref[...], preferred_element_type=jnp.float32)
    m = jnp.mean(h, axis=1, keepdims=True)
    v = jnp.mean((h - m) * (h - m), axis=1, keepdims=True)
    h = (h - m) * lax.rsqrt(v + 1e-5) * g_ref[...] + b_ref[...]
    o_ref[...] = _leaky(h)


def _post_tc(x, w1, w2, g, b):
    k = x.shape[1]
    return pl.pallas_call(
        _post_body,
        grid=(N // RB,),
        in_specs=[
            pl.BlockSpec((RB, k), lambda i: (i, 0)),
            pl.BlockSpec((k, HID), lambda i: (0, 0)),
            pl.BlockSpec((HID, HID), lambda i: (0, 0)),
            pl.BlockSpec((1, HID), lambda i: (0, 0)),
            pl.BlockSpec((1, HID), lambda i: (0, 0)),
        ],
        out_specs=pl.BlockSpec((RB, HID), lambda i: (i, 0)),
        out_shape=jax.ShapeDtypeStruct((N, HID), jnp.float32),
    )(x, w1, w2, g.reshape(1, HID), b.reshape(1, HID))


def _sage2_body(sa_ref, ca_ref, sb_ref, cb_ref, h_ref,
                wla_ref, wlb_ref, wr_ref, bb_ref, o_ref):
    ra = 1.0 / jnp.maximum(ca_ref[...][:, 0:1], 1.0)
    rb = 1.0 / jnp.maximum(cb_ref[...][:, 0:1], 1.0)
    acc = jnp.dot(h_ref[...], wr_ref[...], preferred_element_type=jnp.float32)
    acc += jnp.dot(sa_ref[0] * ra, wla_ref[0],
                   preferred_element_type=jnp.float32)
    acc += jnp.dot(sa_ref[1] * ra, wla_ref[1],
                   preferred_element_type=jnp.float32)
    acc += jnp.dot(sb_ref[0] * rb, wlb_ref[0],
                   preferred_element_type=jnp.float32)
    acc += jnp.dot(sb_ref[1] * rb, wlb_ref[1],
                   preferred_element_type=jnp.float32)
    o_ref[...] = _leaky(acc + bb_ref[...])


def _sage2_tc(sa, ca, sb, cb, h, wla, wlb, wr, bb):
    return pl.pallas_call(
        _sage2_body,
        grid=(N // RB,),
        in_specs=[
            pl.BlockSpec((NC, RB, 128), lambda i: (0, i, 0)),
            pl.BlockSpec((RB, 16), lambda i: (i, 0)),
            pl.BlockSpec((NC, RB, 128), lambda i: (0, i, 0)),
            pl.BlockSpec((RB, 16), lambda i: (i, 0)),
            pl.BlockSpec((RB, HID), lambda i: (i, 0)),
            pl.BlockSpec((NC, 128, HID), lambda i: (0, 0, 0)),
            pl.BlockSpec((NC, 128, HID), lambda i: (0, 0, 0)),
            pl.BlockSpec((HID, HID), lambda i: (0, 0)),
            pl.BlockSpec((1, HID), lambda i: (0, 0)),
        ],
        out_specs=pl.BlockSpec((RB, HID), lambda i: (i, 0)),
        out_shape=jax.ShapeDtypeStruct((N, HID), jnp.float32),
    )(sa, ca, sb, cb, h, wla, wlb, wr, bb)


def _sage1_body(sa_ref, ca_ref, h_ref, wla_ref, wr_ref, bb_ref, o_ref):
    ra = 1.0 / jnp.maximum(ca_ref[...][:, 0:1], 1.0)
    acc = jnp.dot(h_ref[...], wr_ref[...], preferred_element_type=jnp.float32)
    acc += jnp.dot(sa_ref[0] * ra, wla_ref[0],
                   preferred_element_type=jnp.float32)
    acc += jnp.dot(sa_ref[1] * ra, wla_ref[1],
                   preferred_element_type=jnp.float32)
    o_ref[...] = _leaky(acc + bb_ref[...])


def _sage1_tc(sa, ca, h, wla, wr, bb):
    return pl.pallas_call(
        _sage1_body,
        grid=(N // RB,),
        in_specs=[
            pl.BlockSpec((NC, RB, 128), lambda i: (0, i, 0)),
            pl.BlockSpec((RB, 16), lambda i: (i, 0)),
            pl.BlockSpec((RB, HID), lambda i: (i, 0)),
            pl.BlockSpec((NC, 128, HID), lambda i: (0, 0, 0)),
            pl.BlockSpec((HID, HID), lambda i: (0, 0)),
            pl.BlockSpec((1, HID), lambda i: (0, 0)),
        ],
        out_specs=pl.BlockSpec((RB, HID), lambda i: (i, 0)),
        out_shape=jax.ShapeDtypeStruct((N, HID), jnp.float32),
    )(sa, ca, h, wla, wr, bb)


@jax.jit
def kernel(x_target, x_reference, edge_index_tt, edge_index_rr,
           edge_index_rt, params):
    p = params
    e_tt = _prep_edges(edge_index_tt)
    e_rr = _prep_edges(edge_index_rr)
    e_rt = _prep_edges(edge_index_rt)
    cnt_tt, cnt_rr, cnt_rt = _count3(e_tt[1], e_rr[1], e_rt[1])

    h_tgt = _post_tc(x_target, p['W_win'], p['W_post'], p['ln_g'], p['ln_b'])
    h_ref = _post_tc(x_reference, p['W_exp'], p['W_post'], p['ln_g'], p['ln_b'])

    for layer in p['layers']:
        seg_tt = _segsum(h_tgt, *e_tt)
        seg_rr = _segsum(h_ref, *e_rr)
        seg_rt = _segsum(h_ref, *e_rt)

        wla = (0.5 * layer['Wl_tt']).reshape(NC, 128, HID)
        wlb = (0.5 * layer['Wl_rt']).reshape(NC, 128, HID)
        wr = 0.5 * (layer['Wr_tt'] + layer['Wr_rt'])
        bb = (0.5 * (layer['b_tt'] + layer['b_rt'])).reshape(1, HID)
        h_tgt_new = _sage2_tc(seg_tt, cnt_tt, seg_rt, cnt_rt, h_tgt,
                              wla, wlb, wr, bb)

        wlr = layer['Wl_rr'].reshape(NC, 128, HID)
        h_ref = _sage1_tc(seg_rr, cnt_rr, h_ref, wlr, layer['Wr_rr'],
                          layer['b_rr'].reshape(1, HID))
        h_tgt = h_tgt_new

    return (h_tgt, h_ref)


# speedup vs baseline: 1.8971x; 1.3659x over previous
import functools

import jax
import jax.numpy as jnp
from jax import lax
from jax.experimental import pallas as pl
from jax.experimental.pallas import tpu as pltpu
from jax.experimental.pallas import tpu_sc as plsc

N = 10000
E = 160000
HID = 256
OUT_CH = 64

NC = 2
NS = 16
CK = 128
CP = 79
EPS = CP * CK
EPAD = NS * EPS
NDP = 10112
DRAIN = 624
ZR = NDP // NS


def _leaky(x):
    return jnp.where(x >= 0.0, x, 0.2 * x)


def _segsum_body(h2, src2, dst3, seg_out, sidx, didx, rows, acc, sem):
    c = lax.axis_index("c")
    s = lax.axis_index("s")

    pltpu.sync_copy(src2.at[c, s], sidx)
    pltpu.sync_copy(dst3.at[s], didx)

    def _fill_rows(i, _):
        for k in range(8):
            rows[i, pl.ds(k * 16, 16)] = jnp.zeros((16,), jnp.float32)
        return 0
    lax.fori_loop(0, CK, _fill_rows, 0)

    b0 = s * ZR
    for t in range(4):
        pltpu.sync_copy(rows, acc.at[pl.ds(b0 + t * CK, CK)])
    pltpu.sync_copy(rows.at[pl.ds(0, ZR - 4 * CK)],
                    acc.at[pl.ds(b0 + 4 * CK, ZR - 4 * CK)])
    plsc.subcore_barrier()

    def _step(j, _):
        pltpu.async_copy(h2.at[sidx.at[j]], rows, sem).wait()
        pltpu.sync_copy(rows, acc.at[didx.at[j]], add=True)
        return 0
    lax.fori_loop(0, CP, _step, 0)

    plsc.subcore_barrier()

    d0 = s * DRAIN
    pltpu.sync_copy(acc.at[pl.ds(d0, DRAIN)], seg_out.at[c, pl.ds(d0, DRAIN)])

    tail0 = NS * DRAIN
    tail = N - tail0

    @pl.when(s == NS - 1)
    def _():
        pltpu.sync_copy(acc.at[pl.ds(tail0, tail)],
                        seg_out.at[c, pl.ds(tail0, tail)])


@functools.lru_cache(maxsize=None)
def _segsum_call():
    return pl.kernel(
        _segsum_body,
        out_type=jax.ShapeDtypeStruct((NC, N, 128), jnp.float32),
        mesh=plsc.VectorSubcoreMesh(core_axis_name="c", subcore_axis_name="s",
                                    num_cores=NC, num_subcores=NS),
        scratch_types=[
            pltpu.VMEM((CP, CK), jnp.int32),
            pltpu.VMEM((CP, CK), jnp.int32),
            pltpu.VMEM((CK, 128), jnp.float32),
            pltpu.VMEM_SHARED((NDP, 128), jnp.float32),
            pltpu.SemaphoreType.DMA,
        ],
    )


def _segsum(h, src2, dst3):
    h2 = h.reshape(2 * N, 128)
    return _segsum_call()(h2, src2, dst3)


def _count_body(dstall, cnt_out, didx, onesb, cacc):
    c = lax.axis_index("c")
    s = lax.axis_index("s")

    def _fill(i, _):
        for k in range(8):
            onesb[i, pl.ds(k * 16, 16)] = jnp.zeros((16,), jnp.float32)
        return 0
    lax.fori_loop(0, CK, _fill, 0)

    b0 = s * ZR
    for t in range(4):
        pltpu.sync_copy(onesb, cacc.at[pl.ds(b0 + t * CK, CK)])
    pltpu.sync_copy(onesb.at[pl.ds(0, ZR - 4 * CK)],
                    cacc.at[pl.ds(b0 + 4 * CK, ZR - 4 * CK)])
    plsc.subcore_barrier()

    for k in range(3):
        pltpu.sync_copy(dstall.at[k, s], didx)

        def _lane(i, _, k=k):
            if k > 0:
                onesb[i, pl.ds(16 * (k - 1), 16)] = jnp.zeros((16,),
                                                              jnp.float32)
            onesb[i, pl.ds(16 * k, 16)] = jnp.ones((16,), jnp.float32)
            return 0
        lax.fori_loop(0, CK, _lane, 0)

        def _step(t, _):
            pltpu.sync_copy(onesb, cacc.at[didx.at[2 * t + c]], add=True)
            return 0
        lax.fori_loop(0, CP // 2, _step, 0)

        if CP % 2:
            @pl.when(c == 1)
            def _():
                pltpu.sync_copy(onesb, cacc.at[didx.at[CP - 1]], add=True)

    plsc.subcore_barrier()

    d0 = s * DRAIN
    pltpu.sync_copy(cacc.at[pl.ds(d0, DRAIN)], cnt_out.at[c, pl.ds(d0, DRAIN)])

    tail0 = NS * DRAIN

    @pl.when(s == NS - 1)
    def _():
        pltpu.sync_copy(cacc.at[pl.ds(tail0, N - tail0)],
                        cnt_out.at[c, pl.ds(tail0, N - tail0)])


@functools.lru_cache(maxsize=None)
def _count_call():
    return pl.kernel(
        _count_body,
        out_type=jax.ShapeDtypeStruct((NC, N, 128), jnp.float32),
        mesh=plsc.VectorSubcoreMesh(core_axis_name="c", subcore_axis_name="s",
                                    num_cores=NC, num_subcores=NS),
        scratch_types=[
            pltpu.VMEM((CP, CK), jnp.int32),
            pltpu.VMEM((CK, 128), jnp.float32),
            pltpu.VMEM_SHARED((NDP, 128), jnp.float32),
        ],
    )


def _count3(dst_tt, dst_rr, dst_rt):
    parts = _count_call()(jnp.stack([dst_tt, dst_rr, dst_rt]))
    both = parts[0, :, 0:48] + parts[1, :, 0:48]
    return both[:, 0:16], both[:, 16:32], both[:, 32:48]


def _prep_edges(ei):
    src = ei[0]
    dst = ei[1]
    pad = EPAD - E
    ar = jnp.arange(pad, dtype=jnp.int32)
    srcp = jnp.concatenate([src, ar % N])
    dstp = jnp.concatenate([dst, N + ar % (NDP - N)])
    src2 = jnp.stack([2 * srcp, 2 * srcp + 1]).reshape(NC, NS, CP, CK)
    dst3 = dstp.reshape(NS, CP, CK)
    return src2, dst3


RB = 1000


def _post_body(x_ref, w1_ref, w2_ref, g_ref, b_ref, o_ref):
    h = _leaky(jnp.dot(x_ref[...], w1_ref[...],
                       preferred_element_type=jnp.float32))
    h = jnp.dot(h, w2_ref[...], preferred_element_type=jnp.float32)
    m = jnp.mean(h, axis=1, keepdims=True)
    v = jnp.mean((h - m) * (h - m), axis=1, keepdims=True)
    h = (h - m) * lax.rsqrt(v + 1e-5) * g_ref[...] + b_ref[...]
    o_ref[...] = _leaky(h)


def _post_tc(x, w1, w2, g, b):
    k = x.shape[1]
    return pl.pallas_call(
        _post_body,
        grid=(N // RB,),
        in_specs=[
            pl.BlockSpec((RB, k), lambda i: (i, 0)),
            pl.BlockSpec((k, HID), lambda i: (0, 0)),
            pl.BlockSpec((HID, HID), lambda i: (0, 0)),
            pl.BlockSpec((1, HID), lambda i: (0, 0)),
            pl.BlockSpec((1, HID), lambda i: (0, 0)),
        ],
        out_specs=pl.BlockSpec((RB, HID), lambda i: (i, 0)),
        out_shape=jax.ShapeDtypeStruct((N, HID), jnp.float32),
    )(x, w1, w2, g.reshape(1, HID), b.reshape(1, HID))


def _sage2_body(sa_ref, ca_ref, sb_ref, cb_ref, h_ref,
                wla_ref, wlb_ref, wr_ref, bb_ref, o_ref):
    ra = 1.0 / jnp.maximum(ca_ref[...][:, 0:1], 1.0)
    rb = 1.0 / jnp.maximum(cb_ref[...][:, 0:1], 1.0)
    acc = jnp.dot(h_ref[...], wr_ref[...], preferred_element_type=jnp.float32)
    acc += jnp.dot(sa_ref[0] * ra, wla_ref[0],
                   preferred_element_type=jnp.float32)
    acc += jnp.dot(sa_ref[1] * ra, wla_ref[1],
                   preferred_element_type=jnp.float32)
    acc += jnp.dot(sb_ref[0] * rb, wlb_ref[0],
                   preferred_element_type=jnp.float32)
    acc += jnp.dot(sb_ref[1] * rb, wlb_ref[1],
                   preferred_element_type=jnp.float32)
    o_ref[...] = _leaky(acc + bb_ref[...])


def _sage2_tc(sa, ca, sb, cb, h, wla, wlb, wr, bb):
    return pl.pallas_call(
        _sage2_body,
        grid=(N // RB,),
        in_specs=[
            pl.BlockSpec((NC, RB, 128), lambda i: (0, i, 0)),
            pl.BlockSpec((RB, 16), lambda i: (i, 0)),
            pl.BlockSpec((NC, RB, 128), lambda i: (0, i, 0)),
            pl.BlockSpec((RB, 16), lambda i: (i, 0)),
            pl.BlockSpec((RB, HID), lambda i: (i, 0)),
            pl.BlockSpec((NC, 128, HID), lambda i: (0, 0, 0)),
            pl.BlockSpec((NC, 128, HID), lambda i: (0, 0, 0)),
            pl.BlockSpec((HID, HID), lambda i: (0, 0)),
            pl.BlockSpec((1, HID), lambda i: (0, 0)),
        ],
        out_specs=pl.BlockSpec((RB, HID), lambda i: (i, 0)),
        out_shape=jax.ShapeDtypeStruct((N, HID), jnp.float32),
    )(sa, ca, sb, cb, h, wla, wlb, wr, bb)


def _sage1_body(sa_ref, ca_ref, h_ref, wla_ref, wr_ref, bb_ref, o_ref):
    ra = 1.0 / jnp.maximum(ca_ref[...][:, 0:1], 1.0)
    acc = jnp.dot(h_ref[...], wr_ref[...], preferred_element_type=jnp.float32)
    acc += jnp.dot(sa_ref[0] * ra, wla_ref[0],
                   preferred_element_type=jnp.float32)
    acc += jnp.dot(sa_ref[1] * ra, wla_ref[1],
                   preferred_element_type=jnp.float32)
    o_ref[...] = _leaky(acc + bb_ref[...])


def _sage1_tc(sa, ca, h, wla, wr, bb):
    return pl.pallas_call(
        _sage1_body,
        grid=(N // RB,),
        in_specs=[
            pl.BlockSpec((NC, RB, 128), lambda i: (0, i, 0)),
            pl.BlockSpec((RB, 16), lambda i: (i, 0)),
            pl.BlockSpec((RB, HID), lambda i: (i, 0)),
            pl.BlockSpec((NC, 128, HID), lambda i: (0, 0, 0)),
            pl.BlockSpec((HID, HID), lambda i: (0, 0)),
            pl.BlockSpec((1, HID), lambda i: (0, 0)),
        ],
        out_specs=pl.BlockSpec((RB, HID), lambda i: (i, 0)),
        out_shape=jax.ShapeDtypeStruct((N, HID), jnp.float32),
    )(sa, ca, h, wla, wr, bb)


@jax.jit
def kernel(x_target, x_reference, edge_index_tt, edge_index_rr,
           edge_index_rt, params):
    p = params
    e_tt = _prep_edges(edge_index_tt)
    e_rr = _prep_edges(edge_index_rr)
    e_rt = _prep_edges(edge_index_rt)
    cnt_tt, cnt_rr, cnt_rt = _count3(e_tt[1], e_rr[1], e_rt[1])

    h_tgt = _post_tc(x_target, p['W_win'], p['W_post'], p['ln_g'], p['ln_b'])
    h_ref = _post_tc(x_reference, p['W_exp'], p['W_post'], p['ln_g'], p['ln_b'])

    for layer in p['layers']:
        seg_tt = _segsum(h_tgt, *e_tt)
        seg_rr = _segsum(h_ref, *e_rr)
        seg_rt = _segsum(h_ref, *e_rt)

        wla = (0.5 * layer['Wl_tt']).reshape(NC, 128, HID)
        wlb = (0.5 * layer['Wl_rt']).reshape(NC, 128, HID)
        wr = 0.5 * (layer['Wr_tt'] + layer['Wr_rt'])
        bb = (0.5 * (layer['b_tt'] + layer['b_rt'])).reshape(1, HID)
        h_tgt_new = _sage2_tc(seg_tt, cnt_tt, seg_rt, cnt_rt, h_tgt,
                              wla, wlb, wr, bb)

        wlr = layer['Wl_rr'].reshape(NC, 128, HID)
        h_ref = _sage1_tc(seg_rr, cnt_rr, h_ref, wlr, layer['Wr_rr'],
                          layer['b_rr'].reshape(1, HID))
        h_tgt = h_tgt_new

    return (h_tgt, h_ref)
